# Initial kernel scaffold; baseline (speedup 1.0000x reference)
import functools

import jax
import jax.numpy as jnp
from jax import lax
from jax.experimental import pallas as pl
from jax.experimental.pallas import tpu as pltpu
from jax.experimental.pallas import tpu_sc as plsc

NUM_SEG = 64
D = 128
N_ROWS = 100000
NC, NS = 2, 16
NW = NC * NS
ROWS_PER_W = N_ROWS // NW
CHUNK = 125
N_CHUNKS = ROWS_PER_W // CHUNK
NEG_INF = float("-inf")


def _sc_body(x_hbm, ids_hbm, out_hbm, ids_v, buf, acc, sem):
    c = lax.axis_index("c")
    s = lax.axis_index("s")
    wid = s * NC + c

    pltpu.sync_copy(ids_hbm.at[wid], ids_v)

    def ini(i, carry):
        acc[i, :] = jnp.full((D,), NEG_INF, jnp.float32)
        return carry
    lax.fori_loop(0, NUM_SEG, ini, 0)

    def chunk_body(ck, carry):
        row0 = wid * ROWS_PER_W + ck * CHUNK
        pltpu.async_copy(x_hbm.at[pl.ds(row0, CHUNK), :], buf, sem).wait()

        def row_body(r, carry2):
            seg = ids_v[ck * CHUNK + r]
            for j in range(D // 16):
                sl = pl.ds(j * 16, 16)
                acc[seg, sl] = jnp.maximum(acc[seg, sl], buf[r, sl])
            return carry2
        lax.fori_loop(0, CHUNK, row_body, 0)
        return carry

    lax.fori_loop(0, N_CHUNKS, chunk_body, 0)
    pltpu.sync_copy(acc, out_hbm.at[wid])


def _segment_max_partials(x, ids2d):
    mesh = plsc.VectorSubcoreMesh(core_axis_name="c", subcore_axis_name="s")
    return pl.kernel(
        _sc_body,
        out_type=jax.ShapeDtypeStruct((NW, NUM_SEG, D), jnp.float32),
        mesh=mesh,
        scratch_types=[
            pltpu.VMEM((ROWS_PER_W,), jnp.int32),
            pltpu.VMEM((CHUNK, D), jnp.float32),
            pltpu.VMEM((NUM_SEG, D), jnp.float32),
            pltpu.SemaphoreType.DMA,
        ],
    )(x, ids2d)


def _tc_body(p_ref, wt_ref, b_ref, o_ref):
    h = jnp.max(p_ref[...], axis=0)
    y = jnp.dot(h, wt_ref[...], preferred_element_type=jnp.float32)
    o_ref[...] = jnp.maximum(y + b_ref[...], 0.0)


def _combine_linear(partials, Wt, b2d):
    return pl.pallas_call(
        _tc_body,
        out_shape=jax.ShapeDtypeStruct((NUM_SEG, D), jnp.float32),
    )(partials, Wt, b2d)


def kernel(x, edge_index, batch, W, b):
    del edge_index
    ids2d = batch.astype(jnp.int32).reshape(NW, ROWS_PER_W)
    partials = _segment_max_partials(x, ids2d)
    return _combine_linear(partials, W.T, b.reshape(1, D))


# speedup vs baseline: 1.8614x; 1.8614x over previous
import functools

import jax
import jax.numpy as jnp
from jax import lax
from jax.experimental import pallas as pl
from jax.experimental.pallas import tpu as pltpu
from jax.experimental.pallas import tpu_sc as plsc

NUM_SEG = 64
D = 128
N_ROWS = 100000
NC, NS = 2, 16
NW = NC * NS
ROWS_PER_W = N_ROWS // NW
CHUNK = 125
N_CHUNKS = ROWS_PER_W // CHUNK
NEG_INF = float("-inf")


def _sc_body(x_hbm, ids_hbm, out_hbm, ids_v, buf, acc, sem):
    c = lax.axis_index("c")
    s = lax.axis_index("s")
    wid = s * NC + c

    pltpu.sync_copy(ids_hbm.at[wid], ids_v.at[pl.ds(0, ROWS_PER_W)])

    def ini(i, carry):
        acc[i, :] = jnp.full((D,), NEG_INF, jnp.float32)
        return carry
    lax.fori_loop(0, NUM_SEG, ini, 0)

    def chunk_body(ck, carry):
        row0 = wid * ROWS_PER_W + ck * CHUNK
        pltpu.async_copy(x_hbm.at[pl.ds(row0, CHUNK), :], buf, sem).wait()

        def row_body(r, carry2):
            seg = ids_v[pl.ds(ck * CHUNK + r, 16)][0]
            for j in range(D // 16):
                sl = pl.ds(j * 16, 16)
                acc[seg, sl] = jnp.maximum(acc[seg, sl], buf[r, sl])
            return carry2
        lax.fori_loop(0, CHUNK, row_body, 0)
        return carry

    lax.fori_loop(0, N_CHUNKS, chunk_body, 0)
    pltpu.sync_copy(acc, out_hbm.at[wid])


def _segment_max_partials(x, ids2d):
    mesh = plsc.VectorSubcoreMesh(core_axis_name="c", subcore_axis_name="s")
    return pl.kernel(
        _sc_body,
        out_type=jax.ShapeDtypeStruct((NW, NUM_SEG, D), jnp.float32),
        mesh=mesh,
        compiler_params=pltpu.CompilerParams(use_tc_tiling_on_sc=False),
        scratch_types=[
            pltpu.VMEM((ROWS_PER_W + 16,), jnp.int32),
            pltpu.VMEM((CHUNK, D), jnp.float32),
            pltpu.VMEM((NUM_SEG, D), jnp.float32),
            pltpu.SemaphoreType.DMA,
        ],
    )(x, ids2d)


def _tc_body(p_ref, wt_ref, b_ref, o_ref):
    h = jnp.max(p_ref[...], axis=0)
    y = jnp.dot(h, wt_ref[...], preferred_element_type=jnp.float32)
    o_ref[...] = jnp.maximum(y + b_ref[...], 0.0)


def _combine_linear(partials, Wt, b2d):
    return pl.pallas_call(
        _tc_body,
        out_shape=jax.ShapeDtypeStruct((NUM_SEG, D), jnp.float32),
    )(partials, Wt, b2d)


def kernel(x, edge_index, batch, W, b):
    del edge_index
    ids2d = batch.astype(jnp.int32).reshape(NW, ROWS_PER_W)
    partials = _segment_max_partials(x, ids2d)
    return _combine_linear(partials, W.T, b.reshape(1, D))


# speedup vs baseline: 6.3187x; 3.3946x over previous
import jax
import jax.numpy as jnp
from jax import lax
from jax.experimental import pallas as pl
from jax.experimental.pallas import tpu as pltpu
from jax.experimental.pallas import tpu_sc as plsc

NUM_SEG = 64
D = 128
N_ROWS = 100000
NC, NS = 2, 16
NW = NC * NS
CHUNK_R = 128
TOTAL_CHUNKS = -(-N_ROWS // CHUNK_R)
BASE_CK = TOTAL_CHUNKS // NW
EXTRA = TOTAL_CHUNKS - BASE_CK * NW
SLOTS = 2 * (-(-(BASE_CK + 1) // 2))
IDS_SLAB = SLOTS * CHUNK_R
NEG_INF = float("-inf")


def _sc_body(x_hbm, ids_hbm, out_hbm, idsv, bufx0, bufx1, acc, sem0, sem1):
    cc = lax.axis_index("c")
    ss = lax.axis_index("s")
    wid = ss * NC + cc

    nck = jnp.where(wid < EXTRA, BASE_CK + 1, BASE_CK)
    base_ck = wid * BASE_CK + jnp.minimum(wid, EXTRA)
    id_base = jnp.minimum(base_ck * CHUNK_R, N_ROWS - IDS_SLAB)

    pltpu.sync_copy(ids_hbm.at[pl.ds(id_base, IDS_SLAB)],
                    idsv.at[pl.ds(0, IDS_SLAB)])

    def ini(i, carry):
        acc[i, :] = jnp.full((D,), NEG_INF, jnp.float32)
        return carry
    lax.fori_loop(0, NUM_SEG, ini, 0)

    def cstart(cslot):
        g = base_ck + jnp.minimum(cslot, nck - 1)
        return jnp.minimum(g * CHUNK_R, N_ROWS - CHUNK_R)

    def issue(cslot, bufx, sem):
        pltpu.async_copy(x_hbm.at[pl.ds(cstart(cslot), CHUNK_R), :], bufx, sem)

    def wait(bufx, sem):
        pltpu.make_async_copy(x_hbm.at[pl.ds(0, CHUNK_R), :], bufx, sem).wait()

    def process(bufx, cslot):
        off = cstart(cslot) - id_base

        def group_body(gi, carry):
            o = off + gi * 16
            idv = idsv[pl.ds(o, 16)]
            i0 = idv[0]
            i15 = idv[15]

            def fast(_):
                for j in range(D // 16):
                    sl = pl.ds(j * 16, 16)
                    vals = [bufx[gi * 16 + k, sl] for k in range(16)]
                    while len(vals) > 1:
                        vals = [jnp.maximum(vals[2 * t], vals[2 * t + 1])
                                for t in range(len(vals) // 2)]
                    acc[i0, sl] = jnp.maximum(acc[i0, sl], vals[0])
                return 0

            def slow(_):
                def rb(k, c2):
                    seg = idsv[pl.ds(o + k, 16)][0]
                    for j in range(D // 16):
                        sl = pl.ds(j * 16, 16)
                        acc[seg, sl] = jnp.maximum(acc[seg, sl],
                                                   bufx[gi * 16 + k, sl])
                    return c2
                return lax.fori_loop(0, 16, rb, 0)

            lax.cond(i0 == i15, fast, slow, 0)
            return carry

        lax.fori_loop(0, CHUNK_R // 16, group_body, 0)

    issue(0, bufx0, sem0)

    def pair(p, carry):
        c0 = 2 * p
        issue(c0 + 1, bufx1, sem1)
        wait(bufx0, sem0)
        process(bufx0, c0)
        issue(c0 + 2, bufx0, sem0)
        wait(bufx1, sem1)
        process(bufx1, c0 + 1)
        return carry

    lax.fori_loop(0, SLOTS // 2, pair, 0)
    wait(bufx0, sem0)
    pltpu.sync_copy(acc, out_hbm.at[wid])


def _segment_max_partials(x, ids):
    mesh = plsc.VectorSubcoreMesh(core_axis_name="c", subcore_axis_name="s")
    return pl.kernel(
        _sc_body,
        out_type=jax.ShapeDtypeStruct((NW, NUM_SEG, D), jnp.float32),
        mesh=mesh,
        compiler_params=pltpu.CompilerParams(use_tc_tiling_on_sc=False),
        scratch_types=[
            pltpu.VMEM((IDS_SLAB + 16,), jnp.int32),
            pltpu.VMEM((CHUNK_R, D), jnp.float32),
            pltpu.VMEM((CHUNK_R, D), jnp.float32),
            pltpu.VMEM((NUM_SEG, D), jnp.float32),
            pltpu.SemaphoreType.DMA,
            pltpu.SemaphoreType.DMA,
        ],
    )(x, ids)


def _tc_body(p_ref, wt_ref, b_ref, o_ref):
    h = jnp.max(p_ref[...], axis=0)
    y = jnp.dot(h, wt_ref[...], preferred_element_type=jnp.float32)
    o_ref[...] = jnp.maximum(y + b_ref[...], 0.0)


def _combine_linear(partials, Wt, b2d):
    return pl.pallas_call(
        _tc_body,
        out_shape=jax.ShapeDtypeStruct((NUM_SEG, D), jnp.float32),
    )(partials, Wt, b2d)


def kernel(x, edge_index, batch, W, b):
    del edge_index
    partials = _segment_max_partials(x, batch.astype(jnp.int32))
    return _combine_linear(partials, W.T, b.reshape(1, D))


# speedup vs baseline: 6.5308x; 1.0336x over previous
import jax
import jax.numpy as jnp
from jax import lax
from jax.experimental import pallas as pl
from jax.experimental.pallas import tpu as pltpu
from jax.experimental.pallas import tpu_sc as plsc

NUM_SEG = 64
D = 128
N_ROWS = 100000
NC, NS = 2, 16
NW = NC * NS
CHUNK_R = 128
TOTAL_CHUNKS = -(-N_ROWS // CHUNK_R)
BASE_CK = TOTAL_CHUNKS // NW
EXTRA = TOTAL_CHUNKS - BASE_CK * NW
SLOTS = 2 * (-(-(BASE_CK + 1) // 2))
IDS_SLAB = SLOTS * CHUNK_R
NEG_INF = float("-inf")


def _sc_body(x_hbm, ids_hbm, out_hbm, idsv, bufx0, bufx1, acc, sem0, sem1):
    cc = lax.axis_index("c")
    ss = lax.axis_index("s")
    wid = ss * NC + cc

    nck = jnp.where(wid < EXTRA, BASE_CK + 1, BASE_CK)
    base_ck = wid * BASE_CK + jnp.minimum(wid, EXTRA)
    id_base = jnp.minimum(base_ck * CHUNK_R, N_ROWS - IDS_SLAB)

    pltpu.sync_copy(ids_hbm.at[pl.ds(id_base, IDS_SLAB)],
                    idsv.at[pl.ds(0, IDS_SLAB)])

    def ini(i, carry):
        for j in range(D // 16):
            acc[i, pl.ds(j * 16, 16)] = jnp.full((16,), NEG_INF, jnp.float32)
        return carry
    lax.fori_loop(0, NUM_SEG, ini, 0)

    def cstart(cslot):
        g = base_ck + jnp.minimum(cslot, nck - 1)
        return jnp.minimum(g * CHUNK_R, N_ROWS - CHUNK_R)

    def issue(cslot, bufx, sem):
        pltpu.async_copy(x_hbm.at[pl.ds(cstart(cslot), CHUNK_R), :], bufx, sem)

    def wait(bufx, sem):
        pltpu.make_async_copy(x_hbm.at[pl.ds(0, CHUNK_R), :], bufx, sem).wait()

    lane = lax.iota(jnp.int32, 16)

    def process(bufx, cslot):
        off = cstart(cslot) - id_base
        idv0 = idsv[pl.ds(off, 16)]
        idvl = idsv[pl.ds(off + CHUNK_R - 16, 16)]

        def fast_chunk(_):
            neg = jnp.full((16,), NEG_INF, jnp.float32)

            def gb(g, m):
                out = list(m)
                for j in range(D // 16):
                    sl = pl.ds(j * 16, 16)
                    vals = [bufx[g * 16 + k, sl] for k in range(16)]
                    while len(vals) > 1:
                        vals = [jnp.maximum(vals[2 * t], vals[2 * t + 1])
                                for t in range(len(vals) // 2)]
                    out[j] = jnp.maximum(out[j], vals[0])
                return tuple(out)

            red = lax.fori_loop(0, CHUNK_R // 16, gb, (neg,) * (D // 16))
            for j in range(D // 16):
                colv = lane + (j * 16)
                cur = plsc.load_gather(acc, [idv0, colv])
                plsc.store_scatter(acc, [idv0, colv],
                                   jnp.maximum(cur, red[j]))
            return 0

        def slow_chunk(_):
            def group_body(gi, carry):
                o = off + gi * 16
                idv = idsv[pl.ds(o, 16)]
                i0 = idv[0]
                i15 = idv[15]

                def fast(_):
                    for j in range(D // 16):
                        sl = pl.ds(j * 16, 16)
                        vals = [bufx[gi * 16 + k, sl] for k in range(16)]
                        while len(vals) > 1:
                            vals = [jnp.maximum(vals[2 * t], vals[2 * t + 1])
                                    for t in range(len(vals) // 2)]
                        acc[i0, sl] = jnp.maximum(acc[i0, sl], vals[0])
                    return 0

                def slow(_):
                    def rb(k, c2):
                        seg = idsv[pl.ds(o + k, 16)][0]
                        for j in range(D // 16):
                            sl = pl.ds(j * 16, 16)
                            acc[seg, sl] = jnp.maximum(acc[seg, sl],
                                                       bufx[gi * 16 + k, sl])
                        return c2
                    return lax.fori_loop(0, 16, rb, 0)

                lax.cond(i0 == i15, fast, slow, 0)
                return carry

            return lax.fori_loop(0, CHUNK_R // 16, group_body, 0)

        lax.cond(idv0[0] == idvl[15], fast_chunk, slow_chunk, 0)

    issue(0, bufx0, sem0)

    def pair(p, carry):
        c0 = 2 * p
        issue(c0 + 1, bufx1, sem1)
        wait(bufx0, sem0)
        process(bufx0, c0)
        issue(c0 + 2, bufx0, sem0)
        wait(bufx1, sem1)
        process(bufx1, c0 + 1)
        return carry

    lax.fori_loop(0, SLOTS // 2, pair, 0)
    wait(bufx0, sem0)
    pltpu.sync_copy(acc, out_hbm.at[wid])


def _segment_max_partials(x, ids):
    mesh = plsc.VectorSubcoreMesh(core_axis_name="c", subcore_axis_name="s")
    return pl.kernel(
        _sc_body,
        out_type=jax.ShapeDtypeStruct((NW, NUM_SEG, D), jnp.float32),
        mesh=mesh,
        compiler_params=pltpu.CompilerParams(use_tc_tiling_on_sc=False,
                                             needs_layout_passes=False),
        scratch_types=[
            pltpu.VMEM((IDS_SLAB + 16,), jnp.int32),
            pltpu.VMEM((CHUNK_R, D), jnp.float32),
            pltpu.VMEM((CHUNK_R, D), jnp.float32),
            pltpu.VMEM((NUM_SEG, D), jnp.float32),
            pltpu.SemaphoreType.DMA,
            pltpu.SemaphoreType.DMA,
        ],
    )(x, ids)


def _tc_body(p_ref, w_ref, b_ref, o_ref):
    h = jnp.max(p_ref[...], axis=0)
    y = lax.dot_general(h, w_ref[...], (((1,), (1,)), ((), ())),
                        preferred_element_type=jnp.float32)
    o_ref[...] = jnp.maximum(y + b_ref[...], 0.0)


def _combine_linear(partials, W, b2d):
    return pl.pallas_call(
        _tc_body,
        out_shape=jax.ShapeDtypeStruct((NUM_SEG, D), jnp.float32),
    )(partials, W, b2d)


def kernel(x, edge_index, batch, W, b):
    del edge_index
    partials = _segment_max_partials(x, batch.astype(jnp.int32))
    return _combine_linear(partials, W, b.reshape(1, D))
